# baseline (speedup 1.0000x reference)
import functools

import jax
import jax.numpy as jnp
from jax import lax
from jax.experimental import pallas as pl
from jax.experimental.pallas import tpu as pltpu

_H = 32
_LAYERS = 2
_OCT = 8


def _round_up(x, m):
    return ((x + m - 1) // m) * m


def _lstm_cell_t(tg, c_prev, H):
    ti = tg[0 * H:1 * H]
    tf = tg[1 * H:2 * H]
    gg = tg[2 * H:3 * H]
    to = tg[3 * H:4 * H]
    c_new = 0.5 * ((tf + 1.0) * c_prev + (ti + 1.0) * gg)
    h_new = (0.5 * (to + 1.0)) * jnp.tanh(c_new)
    return h_new, c_new


def _rnn_kernel(tok_ref, wx0_ref, wbig_ref, b1_ref,
                w8_ref, bfc_ref, h0_ref, c0_ref,
                logits_ref, hN_ref, cN_ref,
                xg_scr, h0_scr, c0_scr, h1_scr, c1_scr,
                *, Tc, H, V):
    t = pl.program_id(0)
    Bp = h0_scr.shape[1]
    H4 = 4 * H

    @pl.when(t == 0)
    def _():
        h0_scr[...] = h0_ref[0]
        c0_scr[...] = c0_ref[0]
        h1_scr[...] = h0_ref[1]
        c1_scr[...] = c0_ref[1]

    rows = Tc * Bp
    tok = tok_ref[0]
    oh = (lax.broadcasted_iota(jnp.int32, (V, rows), 0) == tok).astype(jnp.bfloat16)
    xg_scr[...] = jnp.dot(wx0_ref[...], oh, preferred_element_type=jnp.float32)

    wbig = wbig_ref[...]
    b1 = b1_ref[...]
    w8 = w8_ref[...]
    bfc = bfc_ref[...]
    w0cat = wbig[:, :H]
    whh1 = wbig[H4:, H:]

    h0v = h0_scr[...]
    c0v = c0_scr[...]
    h1v = h1_scr[...]
    c1v = c1_scr[...]

    a = jnp.dot(w0cat, h0v, preferred_element_type=jnp.float32)
    tg0 = jnp.tanh(a[:H4] + xg_scr[:, pl.ds(0, Bp)])
    h0v, c0v = _lstm_cell_t(tg0, c0v, H)
    a = jnp.dot(w0cat, h0v, preferred_element_type=jnp.float32)
    b = jnp.dot(whh1, h1v, preferred_element_type=jnp.float32)

    oct = []

    def flush(oct, kq):
        octc = jnp.concatenate(oct, axis=1).astype(jnp.bfloat16)
        lg = (jnp.dot(octc, w8, preferred_element_type=jnp.float32) + bfc)
        g0c = (kq - _OCT + 1) * V
        logits_ref[0, :, g0c:g0c + _OCT * V] = lg

    for k in range(1, Tc):
        tg0 = jnp.tanh(a[:H4] + xg_scr[:, k * Bp:(k + 1) * Bp])
        h0v, c0n = _lstm_cell_t(tg0, c0v, H)
        an = jnp.dot(w0cat, h0v, preferred_element_type=jnp.float32)
        tg1 = jnp.tanh(a[H4:] + b + b1)
        h1n, c1v = _lstm_cell_t(tg1, c1v, H)
        bn = jnp.dot(whh1, h1n, preferred_element_type=jnp.float32)
        oct.append(h1n.T)
        if len(oct) == _OCT:
            flush(oct, k - 1)
            oct = []
        a, b, c0v, h1v = an, bn, c0n, h1n

    tg1 = jnp.tanh(a[H4:] + b + b1)
    h1v, c1v = _lstm_cell_t(tg1, c1v, H)
    oct.append(h1v.T)
    flush(oct, Tc - 1)

    h0_scr[...] = h0v
    c0_scr[...] = c0v
    h1_scr[...] = h1v
    c1_scr[...] = c1v

    hN_ref[0] = h0v
    hN_ref[1] = h1v
    cN_ref[0] = c0v
    cN_ref[1] = c1v


def _rnn_call(tok3, wx0, wbig, b1t, w8, bfc8, h0, c0,
              *, Tc, Bp, H, V):
    n_chunks = tok3.shape[0]
    rows = Tc * Bp
    T = n_chunks * Tc
    H4 = 4 * H
    L = h0.shape[0]

    def const(shape):
        return pl.BlockSpec(shape, lambda t, _n=len(shape): (0,) * _n)

    kernel_fn = functools.partial(_rnn_kernel, Tc=Tc, H=H, V=V)

    out_shapes = (
        jax.ShapeDtypeStruct((n_chunks, Bp, Tc * V), jnp.float32),
        jax.ShapeDtypeStruct((L, H, Bp), jnp.float32),
        jax.ShapeDtypeStruct((L, H, Bp), jnp.float32),
    )

    return pl.pallas_call(
        kernel_fn,
        out_shape=out_shapes,
        grid=(n_chunks,),
        in_specs=[
            pl.BlockSpec((1, 1, rows), lambda t: (t, 0, 0)),
            const((H4, V)),
            const((2 * H4, 2 * H)),
            const((H4, Bp)),
            const((_OCT * H, _OCT * V)),
            const((1, _OCT * V)),
            const((L, H, Bp)),
            const((L, H, Bp)),
        ],
        out_specs=[
            pl.BlockSpec((1, Bp, Tc * V), lambda t: (t, 0, 0)),
            const((L, H, Bp)),
            const((L, H, Bp)),
        ],
        scratch_shapes=[
            pltpu.VMEM((H4, rows), jnp.float32),
            pltpu.VMEM((H, Bp), jnp.float32),
            pltpu.VMEM((H, Bp), jnp.float32),
            pltpu.VMEM((H, Bp), jnp.float32),
            pltpu.VMEM((H, Bp), jnp.float32),
        ],
        compiler_params=pltpu.CompilerParams(
            dimension_semantics=("arbitrary",),
            vmem_limit_bytes=100 << 20),
    )(tok3, wx0, wbig, b1t, w8, bfc8, h0, c0)


def kernel(embedding, fc_w, fc_b, w_ih_0, w_hh_0, b_0,
           w_ih_1, w_hh_1, b_1, x_tokens, h0, c0):
    B, T = x_tokens.shape
    H = _H
    V = fc_w.shape[1]
    H4 = 4 * H

    Bp = _round_up(B, 8)
    Tc = 64
    while T % (Tc * _OCT) or Tc % _OCT:
        Tc //= 2
    n_chunks = T // Tc
    rows = Tc * Bp

    scale = jnp.concatenate([
        jnp.full((2 * H,), 0.5, jnp.float32),
        jnp.ones((H,), jnp.float32),
        jnp.full((H,), 0.5, jnp.float32),
    ])[None, :]

    wx0 = (jnp.dot(embedding, w_ih_0 * scale) + b_0 * scale).T.astype(jnp.bfloat16)
    whh0 = (w_hh_0 * scale).T.astype(jnp.float32)
    wih1 = (w_ih_1 * scale).T.astype(jnp.float32)
    whh1 = (w_hh_1 * scale).T.astype(jnp.float32)
    wbig = jnp.concatenate([
        jnp.concatenate([whh0, jnp.zeros((H4, H), jnp.float32)], axis=1),
        jnp.concatenate([wih1, whh1], axis=1),
    ], axis=0)
    b1t = jnp.tile((b_1 * scale).reshape(H4, 1), (1, Bp)).astype(jnp.float32)
    w8 = jnp.kron(jnp.eye(_OCT, dtype=jnp.float32),
                  fc_w).astype(jnp.bfloat16)
    bfc8 = jnp.tile(fc_b.reshape(1, V), (1, _OCT)).astype(jnp.float32)

    tok_t = x_tokens.T
    if Bp != B:
        tok_t = jnp.pad(tok_t, ((0, 0), (0, Bp - B)))
    tok3 = tok_t.reshape(n_chunks, 1, rows)
    h0_p = h0.astype(jnp.float32)
    c0_p = c0.astype(jnp.float32)
    if Bp != B:
        h0_p = jnp.pad(h0_p, ((0, 0), (0, Bp - B), (0, 0)))
        c0_p = jnp.pad(c0_p, ((0, 0), (0, Bp - B), (0, 0)))
    h0_t = h0_p.transpose(0, 2, 1)
    c0_t = c0_p.transpose(0, 2, 1)

    logits3, hN_t, cN_t = _rnn_call(
        tok3, wx0, wbig, b1t, w8, bfc8, h0_t, c0_t,
        Tc=Tc, Bp=Bp, H=H, V=V)

    logits = (logits3.reshape(n_chunks, Bp, Tc, V).transpose(1, 0, 2, 3)
              .reshape(Bp, T, V)[:B].reshape(B * T, V))
    hN = hN_t.transpose(0, 2, 1)[:, :B, :]
    cN = cN_t.transpose(0, 2, 1)[:, :B, :]
    return logits, (hN, cN)


# speedup vs baseline: 3.7445x; 3.7445x over previous
import functools

import jax
import jax.numpy as jnp
from jax import lax
from jax.experimental import pallas as pl
from jax.experimental.pallas import tpu as pltpu

_H = 32
_LAYERS = 2


def _round_up(x, m):
    return ((x + m - 1) // m) * m


def _lstm_cell_t(tg, c_prev, H):
    ti = tg[0 * H:1 * H]
    tf = tg[1 * H:2 * H]
    gg = tg[2 * H:3 * H]
    to = tg[3 * H:4 * H]
    c_new = 0.5 * ((tf + 1.0) * c_prev + (ti + 1.0) * gg)
    h_new = (0.5 * (to + 1.0)) * jnp.tanh(c_new)
    return h_new, c_new


def _rnn_kernel(tok_ref, wx0_ref, wbig_ref, b1_ref,
                wfc_ref, bfc_ref, h0_ref, c0_ref,
                logits_ref, hN_ref, cN_ref,
                xg_scr, seq_scr, h0_scr, c0_scr, h1_scr, c1_scr,
                *, Tc, H, V):
    t = pl.program_id(0)
    Bp = h0_scr.shape[1]
    H4 = 4 * H
    rows = Tc * Bp

    @pl.when(t == 0)
    def _():
        h0_scr[...] = h0_ref[0]
        c0_scr[...] = c0_ref[0]
        h1_scr[...] = h0_ref[1]
        c1_scr[...] = c0_ref[1]

    tok = tok_ref[0]
    oh = (lax.broadcasted_iota(jnp.int32, (V, rows), 0) == tok).astype(jnp.bfloat16)
    xg_scr[...] = jnp.dot(wx0_ref[...], oh, preferred_element_type=jnp.float32)

    wbig = wbig_ref[...]
    b1 = b1_ref[...]
    w0cat = wbig[:, :H]
    whh1 = wbig[H4:, H:]

    h0v = h0_scr[...]
    c0v = c0_scr[...]
    h1v = h1_scr[...]
    c1v = c1_scr[...]

    a = jnp.dot(w0cat, h0v, preferred_element_type=jnp.float32)
    tg0 = jnp.tanh(a[:H4] + xg_scr[:, pl.ds(0, Bp)])
    h0v, c0v = _lstm_cell_t(tg0, c0v, H)
    a = jnp.dot(w0cat, h0v, preferred_element_type=jnp.float32)
    b = jnp.dot(whh1, h1v, preferred_element_type=jnp.float32)

    def body(k, carry):
        a, b, h0v, c0v, c1v = carry
        r = pl.multiple_of(k * Bp, Bp)
        rp = pl.multiple_of((k - 1) * Bp, Bp)
        tg0 = jnp.tanh(a[:H4] + xg_scr[:, pl.ds(r, Bp)])
        h0n, c0n = _lstm_cell_t(tg0, c0v, H)
        an = jnp.dot(w0cat, h0n, preferred_element_type=jnp.float32)
        tg1 = jnp.tanh(a[H4:] + b + b1)
        h1n, c1n = _lstm_cell_t(tg1, c1v, H)
        bn = jnp.dot(whh1, h1n, preferred_element_type=jnp.float32)
        seq_scr[:, pl.ds(rp, Bp)] = h1n
        return (an, bn, h0n, c0n, c1n)

    a, b, h0v, c0v, c1v = lax.fori_loop(
        1, Tc, body, (a, b, h0v, c0v, c1v), unroll=True)

    tg1 = jnp.tanh(a[H4:] + b + b1)
    h1v, c1v = _lstm_cell_t(tg1, c1v, H)
    seq_scr[:, pl.ds((Tc - 1) * Bp, Bp)] = h1v

    h0_scr[...] = h0v
    c0_scr[...] = c0v
    h1_scr[...] = h1v
    c1_scr[...] = c1v

    lg = lax.dot_general(seq_scr[...].astype(jnp.bfloat16), wfc_ref[...],
                         (((0,), (0,)), ((), ())),
                         preferred_element_type=jnp.float32) + bfc_ref[...]
    logits_ref[...] = lg

    hN_ref[0] = h0v
    hN_ref[1] = h1v
    cN_ref[0] = c0v
    cN_ref[1] = c1v


def _rnn_call(tok3, wx0, wbig, b1t, wfc, bfc, h0, c0,
              *, Tc, Bp, H, V):
    n_chunks = tok3.shape[0]
    rows = Tc * Bp
    T = n_chunks * Tc
    H4 = 4 * H
    L = h0.shape[0]

    def const(shape):
        return pl.BlockSpec(shape, lambda t, _n=len(shape): (0,) * _n)

    kernel_fn = functools.partial(_rnn_kernel, Tc=Tc, H=H, V=V)

    out_shapes = (
        jax.ShapeDtypeStruct((T * Bp, V), jnp.float32),
        jax.ShapeDtypeStruct((L, H, Bp), jnp.float32),
        jax.ShapeDtypeStruct((L, H, Bp), jnp.float32),
    )

    return pl.pallas_call(
        kernel_fn,
        out_shape=out_shapes,
        grid=(n_chunks,),
        in_specs=[
            pl.BlockSpec((1, 1, rows), lambda t: (t, 0, 0)),
            const((H4, V)),
            const((2 * H4, 2 * H)),
            const((H4, Bp)),
            const((H, V)),
            const((1, V)),
            const((L, H, Bp)),
            const((L, H, Bp)),
        ],
        out_specs=[
            pl.BlockSpec((rows, V), lambda t: (t, 0)),
            const((L, H, Bp)),
            const((L, H, Bp)),
        ],
        scratch_shapes=[
            pltpu.VMEM((H4, rows), jnp.float32),
            pltpu.VMEM((H, rows), jnp.float32),
            pltpu.VMEM((H, Bp), jnp.float32),
            pltpu.VMEM((H, Bp), jnp.float32),
            pltpu.VMEM((H, Bp), jnp.float32),
            pltpu.VMEM((H, Bp), jnp.float32),
        ],
        compiler_params=pltpu.CompilerParams(
            dimension_semantics=("arbitrary",),
            vmem_limit_bytes=100 << 20),
    )(tok3, wx0, wbig, b1t, wfc, bfc, h0, c0)


def kernel(embedding, fc_w, fc_b, w_ih_0, w_hh_0, b_0,
           w_ih_1, w_hh_1, b_1, x_tokens, h0, c0):
    B, T = x_tokens.shape
    H = _H
    V = fc_w.shape[1]
    H4 = 4 * H

    Bp = _round_up(B, 8)
    Tc = 64
    while T % Tc:
        Tc //= 2
    n_chunks = T // Tc
    rows = Tc * Bp

    scale = jnp.concatenate([
        jnp.full((2 * H,), 0.5, jnp.float32),
        jnp.ones((H,), jnp.float32),
        jnp.full((H,), 0.5, jnp.float32),
    ])[None, :]

    wx0 = (jnp.dot(embedding, w_ih_0 * scale) + b_0 * scale).T.astype(jnp.bfloat16)
    whh0 = (w_hh_0 * scale).T.astype(jnp.float32)
    wih1 = (w_ih_1 * scale).T.astype(jnp.float32)
    whh1 = (w_hh_1 * scale).T.astype(jnp.float32)
    wbig = jnp.concatenate([
        jnp.concatenate([whh0, jnp.zeros((H4, H), jnp.float32)], axis=1),
        jnp.concatenate([wih1, whh1], axis=1),
    ], axis=0)
    b1t = jnp.tile((b_1 * scale).reshape(H4, 1), (1, Bp)).astype(jnp.float32)
    wfc = fc_w.astype(jnp.bfloat16)
    bfc = fc_b.reshape(1, V).astype(jnp.float32)

    tok_t = x_tokens.T
    if Bp != B:
        tok_t = jnp.pad(tok_t, ((0, 0), (0, Bp - B)))
    tok3 = tok_t.reshape(n_chunks, 1, rows)
    h0_p = h0.astype(jnp.float32)
    c0_p = c0.astype(jnp.float32)
    if Bp != B:
        h0_p = jnp.pad(h0_p, ((0, 0), (0, Bp - B), (0, 0)))
        c0_p = jnp.pad(c0_p, ((0, 0), (0, Bp - B), (0, 0)))
    h0_t = h0_p.transpose(0, 2, 1)
    c0_t = c0_p.transpose(0, 2, 1)

    logits2, hN_t, cN_t = _rnn_call(
        tok3, wx0, wbig, b1t, wfc, bfc, h0_t, c0_t,
        Tc=Tc, Bp=Bp, H=H, V=V)

    logits = (logits2.reshape(T, Bp, V)[:, :B, :]
              .transpose(1, 0, 2).reshape(B * T, V))
    hN = hN_t.transpose(0, 2, 1)[:, :B, :]
    cN = cN_t.transpose(0, 2, 1)[:, :B, :]
    return logits, (hN, cN)


# speedup vs baseline: 3.7777x; 1.0089x over previous
import functools

import jax
import jax.numpy as jnp
from jax import lax
from jax.experimental import pallas as pl
from jax.experimental.pallas import tpu as pltpu

_H = 32
_LAYERS = 2


def _round_up(x, m):
    return ((x + m - 1) // m) * m


def _lstm_cell_t(tg, c_prev, H):
    ti = tg[0 * H:1 * H]
    tf = tg[1 * H:2 * H]
    gg = tg[2 * H:3 * H]
    to = tg[3 * H:4 * H]
    c_new = 0.5 * ((tf + 1.0) * c_prev + (ti + 1.0) * gg)
    h_new = (0.5 * (to + 1.0)) * jnp.tanh(c_new)
    return h_new, c_new


def _rnn_kernel(tok_ref, wx0_ref, wbig_ref, b1_ref,
                wfc_ref, bfc_ref, h0_ref, c0_ref,
                logits_ref, hN_ref, cN_ref,
                xg_scr, seq_scr, h0_scr, c0_scr, h1_scr, c1_scr,
                *, Tc, H, V):
    t = pl.program_id(0)
    Bp = h0_scr.shape[1]
    H4 = 4 * H
    rows = Tc * Bp

    @pl.when(t == 0)
    def _():
        h0_scr[...] = h0_ref[0]
        c0_scr[...] = c0_ref[0]
        h1_scr[...] = h0_ref[1]
        c1_scr[...] = c0_ref[1]

    tok = tok_ref[0]
    oh = (lax.broadcasted_iota(jnp.int32, (V, rows), 0) == tok).astype(jnp.bfloat16)
    xg_scr[...] = jnp.dot(wx0_ref[...], oh, preferred_element_type=jnp.float32)

    wbig = wbig_ref[...]
    b1 = b1_ref[...]
    w0cat = wbig[:, :H]
    whh1 = wbig[H4:, H:]

    h0v = h0_scr[...]
    c0v = c0_scr[...]
    h1v = h1_scr[...]
    c1v = c1_scr[...]

    a = jnp.dot(w0cat, h0v, preferred_element_type=jnp.float32)
    tg0 = jnp.tanh(a[:H4] + xg_scr[:, pl.ds(0, Bp)])
    h0v, c0v = _lstm_cell_t(tg0, c0v, H)
    a = jnp.dot(w0cat, h0v, preferred_element_type=jnp.float32)
    b = jnp.dot(whh1, h1v, preferred_element_type=jnp.float32)

    def body(k, carry):
        a, b, h0v, c0v, c1v = carry
        r = pl.multiple_of(k * Bp, Bp)
        rp = pl.multiple_of((k - 1) * Bp, Bp)
        tg0 = jnp.tanh(a[:H4] + xg_scr[:, pl.ds(r, Bp)])
        h0n, c0n = _lstm_cell_t(tg0, c0v, H)
        an = jnp.dot(w0cat, h0n, preferred_element_type=jnp.float32)
        tg1 = jnp.tanh(a[H4:] + b + b1)
        h1n, c1n = _lstm_cell_t(tg1, c1v, H)
        bn = jnp.dot(whh1, h1n, preferred_element_type=jnp.float32)
        seq_scr[:, pl.ds(rp, Bp)] = h1n
        return (an, bn, h0n, c0n, c1n)

    a, b, h0v, c0v, c1v = lax.fori_loop(
        1, Tc, body, (a, b, h0v, c0v, c1v), unroll=True)

    tg1 = jnp.tanh(a[H4:] + b + b1)
    h1v, c1v = _lstm_cell_t(tg1, c1v, H)
    seq_scr[:, pl.ds((Tc - 1) * Bp, Bp)] = h1v

    h0_scr[...] = h0v
    c0_scr[...] = c0v
    h1_scr[...] = h1v
    c1_scr[...] = c1v

    lg = lax.dot_general(seq_scr[...].astype(jnp.bfloat16), wfc_ref[...],
                         (((0,), (0,)), ((), ())),
                         preferred_element_type=jnp.float32) + bfc_ref[...]
    logits_ref[...] = lg

    hN_ref[0] = h0v
    hN_ref[1] = h1v
    cN_ref[0] = c0v
    cN_ref[1] = c1v


def _rnn_call(tok3, wx0, wbig, b1t, wfc, bfc, h0, c0,
              *, Tc, Bp, H, V):
    n_chunks = tok3.shape[0]
    rows = Tc * Bp
    T = n_chunks * Tc
    H4 = 4 * H
    L = h0.shape[0]

    def const(shape):
        return pl.BlockSpec(shape, lambda t, _n=len(shape): (0,) * _n)

    kernel_fn = functools.partial(_rnn_kernel, Tc=Tc, H=H, V=V)

    out_shapes = (
        jax.ShapeDtypeStruct((T * Bp, V), jnp.float32),
        jax.ShapeDtypeStruct((L, H, Bp), jnp.float32),
        jax.ShapeDtypeStruct((L, H, Bp), jnp.float32),
    )

    return pl.pallas_call(
        kernel_fn,
        out_shape=out_shapes,
        grid=(n_chunks,),
        in_specs=[
            pl.BlockSpec((1, 1, rows), lambda t: (t, 0, 0)),
            const((H4, V)),
            const((2 * H4, 2 * H)),
            const((H4, Bp)),
            const((H, V)),
            const((1, V)),
            const((L, H, Bp)),
            const((L, H, Bp)),
        ],
        out_specs=[
            pl.BlockSpec((rows, V), lambda t: (t, 0)),
            const((L, H, Bp)),
            const((L, H, Bp)),
        ],
        scratch_shapes=[
            pltpu.VMEM((H4, rows), jnp.float32),
            pltpu.VMEM((H, rows), jnp.float32),
            pltpu.VMEM((H, Bp), jnp.float32),
            pltpu.VMEM((H, Bp), jnp.float32),
            pltpu.VMEM((H, Bp), jnp.float32),
            pltpu.VMEM((H, Bp), jnp.float32),
        ],
        compiler_params=pltpu.CompilerParams(
            dimension_semantics=("arbitrary",),
            vmem_limit_bytes=100 << 20),
    )(tok3, wx0, wbig, b1t, wfc, bfc, h0, c0)


def kernel(embedding, fc_w, fc_b, w_ih_0, w_hh_0, b_0,
           w_ih_1, w_hh_1, b_1, x_tokens, h0, c0):
    B, T = x_tokens.shape
    H = _H
    V = fc_w.shape[1]
    H4 = 4 * H

    Bp = _round_up(B, 8)
    Tc = 128
    while T % Tc:
        Tc //= 2
    n_chunks = T // Tc
    rows = Tc * Bp

    scale = jnp.concatenate([
        jnp.full((2 * H,), 0.5, jnp.float32),
        jnp.ones((H,), jnp.float32),
        jnp.full((H,), 0.5, jnp.float32),
    ])[None, :]

    wx0 = (jnp.dot(embedding, w_ih_0 * scale) + b_0 * scale).T.astype(jnp.bfloat16)
    whh0 = (w_hh_0 * scale).T.astype(jnp.float32)
    wih1 = (w_ih_1 * scale).T.astype(jnp.float32)
    whh1 = (w_hh_1 * scale).T.astype(jnp.float32)
    wbig = jnp.concatenate([
        jnp.concatenate([whh0, jnp.zeros((H4, H), jnp.float32)], axis=1),
        jnp.concatenate([wih1, whh1], axis=1),
    ], axis=0)
    b1t = jnp.tile((b_1 * scale).reshape(H4, 1), (1, Bp)).astype(jnp.float32)
    wfc = fc_w.astype(jnp.bfloat16)
    bfc = fc_b.reshape(1, V).astype(jnp.float32)

    tok_t = x_tokens.T
    if Bp != B:
        tok_t = jnp.pad(tok_t, ((0, 0), (0, Bp - B)))
    tok3 = tok_t.reshape(n_chunks, 1, rows)
    h0_p = h0.astype(jnp.float32)
    c0_p = c0.astype(jnp.float32)
    if Bp != B:
        h0_p = jnp.pad(h0_p, ((0, 0), (0, Bp - B), (0, 0)))
        c0_p = jnp.pad(c0_p, ((0, 0), (0, Bp - B), (0, 0)))
    h0_t = h0_p.transpose(0, 2, 1)
    c0_t = c0_p.transpose(0, 2, 1)

    logits2, hN_t, cN_t = _rnn_call(
        tok3, wx0, wbig, b1t, wfc, bfc, h0_t, c0_t,
        Tc=Tc, Bp=Bp, H=H, V=V)

    logits = (logits2.reshape(T, Bp, V)[:, :B, :]
              .transpose(1, 0, 2).reshape(B * T, V))
    hN = hN_t.transpose(0, 2, 1)[:, :B, :]
    cN = cN_t.transpose(0, 2, 1)[:, :B, :]
    return logits, (hN, cN)
